# Initial kernel scaffold; baseline (speedup 1.0000x reference)
import jax
import jax.numpy as jnp
from jax.experimental import pallas as pl


def kernel(Hc, Rc, Rcs, Wp, Rn, Hp, D, W, theta):
    raise NotImplementedError("write your pallas kernel here")


# speedup vs baseline: 70.6397x; 70.6397x over previous
import jax
import jax.numpy as jnp
from jax.experimental import pallas as pl

_NG, _P, _NPC, _FDIM = 10, 7, 23, 70
_N = _P * _NPC


def _fused_kernel(hc_ref, rc_ref, rcs_ref, wp_ref, rn_ref, hp_ref, d_ref,
                  w_ref, theta_ref, wout_ref, hpk_ref):
    hc = hc_ref[0]
    rc = rc_ref[0]
    rcs = rcs_ref[0]
    eye7 = jnp.eye(_P, dtype=jnp.float32)

    hcrc = jnp.dot(hc, rc, preferred_element_type=jnp.float32)
    dh = 0.1 * jnp.sum(hcrc, axis=0, keepdims=True)
    rcs_d = jnp.sum(rcs * eye7, axis=0, keepdims=True)
    del_rc = rc * (dh / rcs_d)
    rct = del_rc + del_rc.T + rc
    ratio = rct / rc
    rp = jnp.where(eye7 > 0.5, 1.0, ratio)

    rowc = jax.lax.broadcasted_iota(jnp.int32, (_N, _P), 0) // _NPC
    colp = jax.lax.broadcasted_iota(jnp.int32, (_N, _P), 1)
    bmat = (rowc == colp).astype(jnp.float32)
    rowp = jax.lax.broadcasted_iota(jnp.int32, (_P, _N), 0)
    colc = jax.lax.broadcasted_iota(jnp.int32, (_P, _N), 1) // _NPC
    bmat_t = (rowp == colc).astype(jnp.float32)
    t = jnp.dot(bmat, rp, preferred_element_type=jnp.float32)
    scale = jnp.dot(t, bmat_t, preferred_element_type=jnp.float32)
    wout_ref[0] = w_ref[0] * scale

    d_diag = jnp.sum(d_ref[0] * jnp.eye(_NPC, dtype=jnp.float32), axis=-1)
    r = jax.lax.rsqrt(d_diag)
    a = wp_ref[0] * rn_ref[0] * r[:, :, None] * r[:, None, :]
    ahp = jax.lax.dot_general(a, hp_ref[0],
                              (((2,), (1,)), ((0,), (0,))),
                              preferred_element_type=jnp.float32)
    hpk = jax.lax.dot_general(ahp, theta_ref[...],
                              (((2,), (0,)), ((), ())),
                              preferred_element_type=jnp.float32)
    hpk_ref[0] = 0.1 * hpk


def kernel(Hc, Rc, Rcs, Wp, Rn, Hp, D, W, theta):
    grid = (_NG,)
    in_specs = [
        pl.BlockSpec((1, _P, _P), lambda i: (i, 0, 0)),
        pl.BlockSpec((1, _P, _P), lambda i: (i, 0, 0)),
        pl.BlockSpec((1, _P, _P), lambda i: (i, 0, 0)),
        pl.BlockSpec((1, _P, _NPC, _NPC), lambda i: (i, 0, 0, 0)),
        pl.BlockSpec((1, _P, _NPC, _NPC), lambda i: (i, 0, 0, 0)),
        pl.BlockSpec((1, _P, _NPC, _FDIM), lambda i: (i, 0, 0, 0)),
        pl.BlockSpec((1, _P, _NPC, _NPC), lambda i: (i, 0, 0, 0)),
        pl.BlockSpec((1, _N, _N), lambda i: (i, 0, 0)),
        pl.BlockSpec((_FDIM, _FDIM), lambda i: (0, 0)),
    ]
    out_specs = [
        pl.BlockSpec((1, _N, _N), lambda i: (i, 0, 0)),
        pl.BlockSpec((1, _P, _NPC, _FDIM), lambda i: (i, 0, 0, 0)),
    ]
    out_shape = [
        jax.ShapeDtypeStruct((_NG, _N, _N), jnp.float32),
        jax.ShapeDtypeStruct((_NG, _P, _NPC, _FDIM), jnp.float32),
    ]
    w_out, hp_k = pl.pallas_call(
        _fused_kernel,
        grid=grid,
        in_specs=in_specs,
        out_specs=out_specs,
        out_shape=out_shape,
    )(Hc, Rc, Rcs, Wp, Rn, Hp, D, W, theta)
    return (w_out, hp_k)
